# baseline (speedup 1.0000x reference)
import jax
import jax.numpy as jnp
from jax import lax
from jax.experimental import pallas as pl
from jax.experimental.pallas import tpu as pltpu
from jax.experimental.pallas import tpu_sc as plsc

N_CORES = 2
N_SUB = 16
NW = N_CORES * N_SUB
L = 16

D = 128
DV = D // L
CHUNK = 80


def _worker_id():
  return lax.axis_index("c") * N_SUB + lax.axis_index("s")


def _zero16():
  return jnp.zeros((L,), jnp.float32)


def _zero_vmem_rows(ref, nrows, width):
  z = _zero16()

  def body(i, carry):
    for k in range(width // L):
      ref[i, pl.ds(L * k, L)] = z
    return carry

  lax.fori_loop(0, nrows, body, 0)


def _fill_flat(ref, n, value):
  v = jnp.full((L,), value, jnp.float32)

  def body(i, carry):
    ref[pl.ds(i * L, L)] = v
    return carry

  lax.fori_loop(0, n // L, body, 0)


def _zero_shared_flat(shared, tmp, base, n, tmp_n):
  done = 0
  while done < n:
    m = min(tmp_n, n - done)
    pltpu.sync_copy(tmp.at[pl.ds(0, m)], shared.at[pl.ds(base + done, m)])
    done += m


def _zero_shared_slice(shared, tmp, base_row, nrows, tmp_rows):
  done = 0
  while done < nrows:
    n = min(tmp_rows, nrows - done)
    pltpu.sync_copy(tmp.at[pl.ds(0, n)],
                    shared.at[pl.ds(base_row + done, n)])
    done += n


def _phase1_body(x_hbm, src_hbm, dst_hbm, hsum_out, deg_out,
                 hsum_sh, deg_sh, sidx_v, didx_v, rows_v, ones_v,
                 semg):
  N = x_hbm.shape[0]
  E = src_hbm.shape[0]
  epw = E // NW
  n_chunks = epw // CHUNK
  rows_per_tile = hsum_sh.shape[0] // N_SUB

  c = lax.axis_index("c")
  s = lax.axis_index("s")
  wid = _worker_id()
  base_row = s * rows_per_tile

  _zero_vmem_rows(rows_v.at[0], CHUNK, D)
  _fill_flat(ones_v, CHUNK, 1.0)
  _zero_shared_slice(hsum_sh, rows_v.at[0], base_row, rows_per_tile, CHUNK)
  _zero_shared_flat(deg_sh, rows_v.at[0, 0], base_row, rows_per_tile, CHUNK)
  plsc.subcore_barrier()

  def fetch(k, slot):
    base = wid * epw + k * CHUNK
    pltpu.sync_copy(src_hbm.at[pl.ds(base, CHUNK)], sidx_v.at[slot])
    pltpu.sync_copy(dst_hbm.at[pl.ds(base, CHUNK)], didx_v.at[slot])
    pltpu.async_copy(x_hbm.at[sidx_v.at[slot]], rows_v.at[slot],
                     semg.at[slot])

  fetch(0, 0)

  def chunk_body(k, carry):
    slot = lax.rem(k, 2)
    nslot = 1 - slot

    @pl.when(k + 1 < n_chunks)
    def _():
      fetch(k + 1, nslot)

    pltpu.make_async_copy(x_hbm.at[pl.ds(0, CHUNK)], rows_v.at[slot],
                          semg.at[slot]).wait()
    pltpu.sync_copy(rows_v.at[slot], hsum_sh.at[didx_v.at[slot]], add=True)
    pltpu.sync_copy(ones_v, deg_sh.at[didx_v.at[slot]], add=True)
    return carry

  lax.fori_loop(0, n_chunks, chunk_body, 0)
  plsc.subcore_barrier()

  pltpu.sync_copy(hsum_sh.at[pl.ds(base_row, rows_per_tile)],
                  hsum_out.at[c, pl.ds(base_row, rows_per_tile)])
  pltpu.sync_copy(deg_sh.at[pl.ds(base_row, rows_per_tile)],
                  deg_out.at[c, pl.ds(base_row, rows_per_tile)])


def _phase2_body(x_hbm, hsrc_hbm, hdst_hbm, src_hbm, dst_hbm, watt_hbm,
                 acc_out, asum_out,
                 acc_sh, asum_sh, sidx_v, didx_v, hs_v, xr_v, arow_v, watt_v,
                 semg, semx, sema):
  N = x_hbm.shape[0]
  E = src_hbm.shape[0]
  epw = E // NW
  n_chunks = epw // CHUNK
  rows_per_tile = acc_sh.shape[0] // N_SUB

  c = lax.axis_index("c")
  s = lax.axis_index("s")
  wid = _worker_id()
  base_row = s * rows_per_tile

  pltpu.sync_copy(watt_hbm, watt_v)
  wv = [watt_v[pl.ds(L * k, L)] for k in range(DV)]

  _zero_vmem_rows(xr_v.at[0], CHUNK, D)
  _fill_flat(arow_v.at[0], CHUNK, 0.0)
  _zero_shared_slice(acc_sh, xr_v.at[0], base_row, rows_per_tile, CHUNK)
  _zero_shared_flat(asum_sh, arow_v.at[0], base_row, rows_per_tile, CHUNK)
  plsc.subcore_barrier()

  lane = lax.iota(jnp.int32, L)

  def fetch(k, slot):
    base = wid * epw + k * CHUNK
    pltpu.sync_copy(src_hbm.at[pl.ds(base, CHUNK)], sidx_v.at[slot])
    pltpu.sync_copy(dst_hbm.at[pl.ds(base, CHUNK)], didx_v.at[slot])
    pltpu.async_copy(hsrc_hbm.at[sidx_v.at[slot]], hs_v.at[slot],
                     semg.at[slot])
    pltpu.async_copy(x_hbm.at[sidx_v.at[slot]], xr_v.at[slot],
                     semx.at[slot])

  fetch(0, 0)

  def chunk_body(k, carry):
    slot = lax.rem(k, 2)
    nslot = 1 - slot

    @pl.when(k + 1 < n_chunks)
    def _():
      fetch(k + 1, nslot)

    pltpu.make_async_copy(x_hbm.at[pl.ds(0, CHUNK)], hs_v.at[slot],
                          semg.at[slot]).wait()
    pltpu.async_copy(hdst_hbm.at[didx_v.at[slot]], hs_v.at[slot],
                     sema.at[slot], add=True)
    pltpu.make_async_copy(x_hbm.at[pl.ds(0, CHUNK)], xr_v.at[slot],
                          semx.at[slot]).wait()
    pltpu.make_async_copy(x_hbm.at[pl.ds(0, CHUNK)], hs_v.at[slot],
                          sema.at[slot]).wait()

    @plsc.parallel_loop(0, CHUNK // L, 1, unroll=2)
    def group_body(g):
      zv = _zero16()
      for ii in range(L):
        i = g * L + ii
        acc = None
        for kk in range(DV):
          e16 = hs_v[slot, i, pl.ds(L * kk, L)]
          lrelu = jnp.maximum(e16, 0.01 * e16)
          t = lrelu * wv[kk]
          acc = t if acc is None else acc + t
        z = jnp.sum(acc)
        zv = jnp.where(lane == ii, z, zv)
      a16 = jnp.exp(zv)
      arow_v[slot, pl.ds(g * L, L)] = a16
      for ii in range(L):
        i = g * L + ii
        av = a16[ii]
        for kk in range(DV):
          xr_v[slot, i, pl.ds(L * kk, L)] = (
              xr_v[slot, i, pl.ds(L * kk, L)] * av)

    pltpu.sync_copy(xr_v.at[slot], acc_sh.at[didx_v.at[slot]], add=True)
    pltpu.sync_copy(arow_v.at[slot], asum_sh.at[didx_v.at[slot]], add=True)
    return carry

  lax.fori_loop(0, n_chunks, chunk_body, 0)
  plsc.subcore_barrier()

  pltpu.sync_copy(acc_sh.at[pl.ds(base_row, rows_per_tile)],
                  acc_out.at[c, pl.ds(base_row, rows_per_tile)])
  pltpu.sync_copy(asum_sh.at[pl.ds(base_row, rows_per_tile)],
                  asum_out.at[c, pl.ds(base_row, rows_per_tile)])


def _proj_body(x_ref, hp_ref, dp_ref, wsrc_ref, bsrc_ref, wdst_ref, bdst_ref,
               hsrc_out, hdst_out):
  n = x_ref.shape[0]
  xb = x_ref[...]
  hp = hp_ref[...]
  dp = dp_ref[...]
  hsum = hp[0, :n] + hp[1, :n]
  deg = dp[0, :n] + dp[1, :n]
  hmean = hsum / jnp.maximum(deg, 1.0)[:, None]
  dims = (((1,), (1,)), ((), ()))
  hsrc_out[...] = (
      lax.dot_general(xb, wsrc_ref[...], dims,
                      preferred_element_type=jnp.float32) + bsrc_ref[...])
  hdst_out[...] = (
      lax.dot_general(hmean, wdst_ref[...], dims,
                      preferred_element_type=jnp.float32) + bdst_ref[...])


def _finalize_body(p_ref, a_ref, out_ref):
  n = out_ref.shape[0]
  p = p_ref[...]
  num = p[0, :n] + p[1, :n]
  a = a_ref[...]
  asum = (a[0, :n] + a[1, :n])[:, None]
  out_ref[...] = jnp.where(asum > 0, num / asum, 0.0)


def kernel(x, edge_index, W_src, b_src, W_dst, b_dst, W_att, b_att):
  del b_att
  N, d = x.shape
  E = edge_index.shape[1]
  assert d == D and E % (NW * CHUNK) == 0
  npad = -(-(N + 1) // (N_SUB * 128)) * (N_SUB * 128)

  src = edge_index[0]
  dst = edge_index[1]
  watt = W_att.reshape(D)

  mesh = plsc.VectorSubcoreMesh(core_axis_name="c", subcore_axis_name="s",
                                num_cores=N_CORES, num_subcores=N_SUB)
  sc_params = pltpu.CompilerParams(needs_layout_passes=False)

  phase1 = pl.kernel(
      _phase1_body,
      out_type=(
          jax.ShapeDtypeStruct((N_CORES, npad, D), jnp.float32),
          jax.ShapeDtypeStruct((N_CORES, npad), jnp.float32),
      ),
      mesh=mesh,
      scratch_types=[
          pltpu.VMEM_SHARED((npad, D), jnp.float32),
          pltpu.VMEM_SHARED((npad,), jnp.float32),
          pltpu.VMEM((2, CHUNK), jnp.int32),
          pltpu.VMEM((2, CHUNK), jnp.int32),
          pltpu.VMEM((2, CHUNK, D), jnp.float32),
          pltpu.VMEM((CHUNK,), jnp.float32),
          pltpu.SemaphoreType.DMA((2,)),
      ],
      compiler_params=sc_params,
  )
  hsum_parts, deg_parts = phase1(x, src, dst)

  proj = pl.pallas_call(
      _proj_body,
      out_shape=[
          jax.ShapeDtypeStruct((N, D), jnp.float32),
          jax.ShapeDtypeStruct((N, D), jnp.float32),
      ],
  )
  h_src, h_dst = proj(x, hsum_parts, deg_parts,
                      W_src, b_src.reshape(1, D), W_dst, b_dst.reshape(1, D))

  phase2 = pl.kernel(
      _phase2_body,
      out_type=(
          jax.ShapeDtypeStruct((N_CORES, npad, D), jnp.float32),
          jax.ShapeDtypeStruct((N_CORES, npad), jnp.float32),
      ),
      mesh=mesh,
      scratch_types=[
          pltpu.VMEM_SHARED((npad, D), jnp.float32),
          pltpu.VMEM_SHARED((npad,), jnp.float32),
          pltpu.VMEM((2, CHUNK), jnp.int32),
          pltpu.VMEM((2, CHUNK), jnp.int32),
          pltpu.VMEM((2, CHUNK, D), jnp.float32),
          pltpu.VMEM((2, CHUNK, D), jnp.float32),
          pltpu.VMEM((2, CHUNK), jnp.float32),
          pltpu.VMEM((D,), jnp.float32),
          pltpu.SemaphoreType.DMA((2,)),
          pltpu.SemaphoreType.DMA((2,)),
          pltpu.SemaphoreType.DMA((2,)),
      ],
      compiler_params=sc_params,
  )
  acc_parts, asum_parts = phase2(x, h_src, h_dst, src, dst, watt)

  finalize = pl.pallas_call(
      _finalize_body,
      out_shape=jax.ShapeDtypeStruct((N, D), jnp.float32),
  )
  return finalize(acc_parts, asum_parts)


# speedup vs baseline: 1.2162x; 1.2162x over previous
import jax
import jax.numpy as jnp
from jax import lax
from jax.experimental import pallas as pl
from jax.experimental.pallas import tpu as pltpu
from jax.experimental.pallas import tpu_sc as plsc

N_CORES = 2
N_SUB = 16
NW = N_CORES * N_SUB
L = 16

D = 128
DV = D // L
CHUNK = 80


def _worker_id():
  return lax.axis_index("c") * N_SUB + lax.axis_index("s")


def _zero16():
  return jnp.zeros((L,), jnp.float32)


def _zero_vmem_rows(ref, nrows, width):
  z = _zero16()

  def body(i, carry):
    for k in range(width // L):
      ref[i, pl.ds(L * k, L)] = z
    return carry

  lax.fori_loop(0, nrows, body, 0)


def _fill_flat(ref, n, value):
  v = jnp.full((L,), value, jnp.float32)

  def body(i, carry):
    ref[pl.ds(i * L, L)] = v
    return carry

  lax.fori_loop(0, n // L, body, 0)


def _zero_shared_flat(shared, tmp, base, n, tmp_n):
  done = 0
  while done < n:
    m = min(tmp_n, n - done)
    pltpu.sync_copy(tmp.at[pl.ds(0, m)], shared.at[pl.ds(base + done, m)])
    done += m


def _zero_shared_slice(shared, tmp, base_row, nrows, tmp_rows):
  done = 0
  while done < nrows:
    n = min(tmp_rows, nrows - done)
    pltpu.sync_copy(tmp.at[pl.ds(0, n)],
                    shared.at[pl.ds(base_row + done, n)])
    done += n


def _phase1_body(x_hbm, src_hbm, dst_hbm, hsum_out, deg_out,
                 hsum_sh, deg_sh, sidx_v, didx_v, rows_v, ones_v,
                 semg):
  N = x_hbm.shape[0]
  E = src_hbm.shape[0]
  epw = E // NW
  n_chunks = epw // CHUNK
  rows_per_tile = hsum_sh.shape[0] // N_SUB

  c = lax.axis_index("c")
  s = lax.axis_index("s")
  wid = _worker_id()
  base_row = s * rows_per_tile

  _zero_vmem_rows(rows_v.at[0], CHUNK, D)
  _fill_flat(ones_v, CHUNK, 1.0)
  _zero_shared_slice(hsum_sh, rows_v.at[0], base_row, rows_per_tile, CHUNK)
  _zero_shared_flat(deg_sh, rows_v.at[0, 0], base_row, rows_per_tile, CHUNK)
  plsc.subcore_barrier()

  def fetch(k, slot):
    base = wid * epw + k * CHUNK
    pltpu.sync_copy(src_hbm.at[pl.ds(base, CHUNK)], sidx_v.at[slot])
    pltpu.sync_copy(dst_hbm.at[pl.ds(base, CHUNK)], didx_v.at[slot])
    pltpu.async_copy(x_hbm.at[sidx_v.at[slot]], rows_v.at[slot],
                     semg.at[slot])

  fetch(0, 0)

  def chunk_body(k, carry):
    slot = lax.rem(k, 2)
    nslot = 1 - slot

    @pl.when(k + 1 < n_chunks)
    def _():
      fetch(k + 1, nslot)

    pltpu.make_async_copy(x_hbm.at[pl.ds(0, CHUNK)], rows_v.at[slot],
                          semg.at[slot]).wait()
    pltpu.sync_copy(rows_v.at[slot], hsum_sh.at[didx_v.at[slot]], add=True)
    pltpu.sync_copy(ones_v, deg_sh.at[didx_v.at[slot]], add=True)
    return carry

  lax.fori_loop(0, n_chunks, chunk_body, 0)
  plsc.subcore_barrier()

  pltpu.sync_copy(hsum_sh.at[pl.ds(base_row, rows_per_tile)],
                  hsum_out.at[c, pl.ds(base_row, rows_per_tile)])
  pltpu.sync_copy(deg_sh.at[pl.ds(base_row, rows_per_tile)],
                  deg_out.at[c, pl.ds(base_row, rows_per_tile)])


def _phase2_body(x_hbm, hsrc_hbm, hdst_hbm, src_hbm, dst_hbm, watt_hbm,
                 acc_out, asum_out,
                 acc_sh, asum_sh, sidx_v, didx_v, hs_v, xr_v, arow_v, watt_v,
                 semg, semx, sema):
  N = x_hbm.shape[0]
  E = src_hbm.shape[0]
  epw = E // NW
  n_chunks = epw // CHUNK
  rows_per_tile = acc_sh.shape[0] // N_SUB

  c = lax.axis_index("c")
  s = lax.axis_index("s")
  wid = _worker_id()
  base_row = s * rows_per_tile

  pltpu.sync_copy(watt_hbm, watt_v)
  wv = [watt_v[pl.ds(L * k, L)] for k in range(DV)]

  _zero_vmem_rows(xr_v.at[0], CHUNK, D)
  _fill_flat(arow_v.at[0], CHUNK, 0.0)
  _zero_shared_slice(acc_sh, xr_v.at[0], base_row, rows_per_tile, CHUNK)
  _zero_shared_flat(asum_sh, arow_v.at[0], base_row, rows_per_tile, CHUNK)
  plsc.subcore_barrier()

  lane = lax.iota(jnp.int32, L)

  def fetch(k, slot):
    base = wid * epw + k * CHUNK
    pltpu.sync_copy(src_hbm.at[pl.ds(base, CHUNK)], sidx_v.at[slot])
    pltpu.sync_copy(dst_hbm.at[pl.ds(base, CHUNK)], didx_v.at[slot])
    pltpu.async_copy(hsrc_hbm.at[sidx_v.at[slot]], hs_v.at[slot],
                     semg.at[slot])
    pltpu.async_copy(x_hbm.at[sidx_v.at[slot]], xr_v.at[slot],
                     semx.at[slot])

  fetch(0, 0)

  def chunk_body(k, carry):
    slot = lax.rem(k, 2)
    nslot = 1 - slot

    pltpu.make_async_copy(x_hbm.at[pl.ds(0, CHUNK)], hs_v.at[slot],
                          semg.at[slot]).wait()
    pltpu.async_copy(hdst_hbm.at[didx_v.at[slot]], hs_v.at[slot],
                     sema.at[slot], add=True)

    @pl.when(k + 1 < n_chunks)
    def _():
      fetch(k + 1, nslot)

    pltpu.make_async_copy(x_hbm.at[pl.ds(0, CHUNK)], xr_v.at[slot],
                          semx.at[slot]).wait()
    pltpu.make_async_copy(x_hbm.at[pl.ds(0, CHUNK)], hs_v.at[slot],
                          sema.at[slot]).wait()

    @plsc.parallel_loop(0, CHUNK // L, 1, unroll=2)
    def group_body(g):
      zv = _zero16()
      for ii in range(L):
        i = g * L + ii
        acc = None
        for kk in range(DV):
          e16 = hs_v[slot, i, pl.ds(L * kk, L)]
          lrelu = jnp.maximum(e16, 0.01 * e16)
          t = lrelu * wv[kk]
          acc = t if acc is None else acc + t
        z = jnp.sum(acc)
        zv = jnp.where(lane == ii, z, zv)
      a16 = jnp.exp(zv)
      arow_v[slot, pl.ds(g * L, L)] = a16
      for ii in range(L):
        i = g * L + ii
        av = a16[ii]
        for kk in range(DV):
          xr_v[slot, i, pl.ds(L * kk, L)] = (
              xr_v[slot, i, pl.ds(L * kk, L)] * av)

    pltpu.sync_copy(xr_v.at[slot], acc_sh.at[didx_v.at[slot]], add=True)
    pltpu.sync_copy(arow_v.at[slot], asum_sh.at[didx_v.at[slot]], add=True)
    return carry

  lax.fori_loop(0, n_chunks, chunk_body, 0)
  plsc.subcore_barrier()

  pltpu.sync_copy(acc_sh.at[pl.ds(base_row, rows_per_tile)],
                  acc_out.at[c, pl.ds(base_row, rows_per_tile)])
  pltpu.sync_copy(asum_sh.at[pl.ds(base_row, rows_per_tile)],
                  asum_out.at[c, pl.ds(base_row, rows_per_tile)])


def _proj_body(x_ref, hp_ref, dp_ref, wsrc_ref, bsrc_ref, wdst_ref, bdst_ref,
               hsrc_out, hdst_out):
  n = x_ref.shape[0]
  xb = x_ref[...]
  hp = hp_ref[...]
  dp = dp_ref[...]
  hsum = hp[0, :n] + hp[1, :n]
  deg = dp[0, :n] + dp[1, :n]
  hmean = hsum / jnp.maximum(deg, 1.0)[:, None]
  dims = (((1,), (1,)), ((), ()))
  hsrc_out[...] = (
      lax.dot_general(xb, wsrc_ref[...], dims,
                      preferred_element_type=jnp.float32) + bsrc_ref[...])
  hdst_out[...] = (
      lax.dot_general(hmean, wdst_ref[...], dims,
                      preferred_element_type=jnp.float32) + bdst_ref[...])


def _finalize_body(p_ref, a_ref, out_ref):
  n = out_ref.shape[0]
  p = p_ref[...]
  num = p[0, :n] + p[1, :n]
  a = a_ref[...]
  asum = (a[0, :n] + a[1, :n])[:, None]
  out_ref[...] = jnp.where(asum > 0, num / asum, 0.0)


def kernel(x, edge_index, W_src, b_src, W_dst, b_dst, W_att, b_att):
  del b_att
  N, d = x.shape
  E = edge_index.shape[1]
  assert d == D and E % (NW * CHUNK) == 0
  npad = -(-(N + 1) // (N_SUB * 128)) * (N_SUB * 128)

  src = edge_index[0]
  dst = edge_index[1]
  watt = W_att.reshape(D)

  mesh = plsc.VectorSubcoreMesh(core_axis_name="c", subcore_axis_name="s",
                                num_cores=N_CORES, num_subcores=N_SUB)
  sc_params = pltpu.CompilerParams(needs_layout_passes=False)

  phase1 = pl.kernel(
      _phase1_body,
      out_type=(
          jax.ShapeDtypeStruct((N_CORES, npad, D), jnp.float32),
          jax.ShapeDtypeStruct((N_CORES, npad), jnp.float32),
      ),
      mesh=mesh,
      scratch_types=[
          pltpu.VMEM_SHARED((npad, D), jnp.float32),
          pltpu.VMEM_SHARED((npad,), jnp.float32),
          pltpu.VMEM((2, CHUNK), jnp.int32),
          pltpu.VMEM((2, CHUNK), jnp.int32),
          pltpu.VMEM((2, CHUNK, D), jnp.float32),
          pltpu.VMEM((CHUNK,), jnp.float32),
          pltpu.SemaphoreType.DMA((2,)),
      ],
      compiler_params=sc_params,
  )
  hsum_parts, deg_parts = phase1(x, src, dst)

  proj = pl.pallas_call(
      _proj_body,
      out_shape=[
          jax.ShapeDtypeStruct((N, D), jnp.float32),
          jax.ShapeDtypeStruct((N, D), jnp.float32),
      ],
  )
  h_src, h_dst = proj(x, hsum_parts, deg_parts,
                      W_src, b_src.reshape(1, D), W_dst, b_dst.reshape(1, D))

  phase2 = pl.kernel(
      _phase2_body,
      out_type=(
          jax.ShapeDtypeStruct((N_CORES, npad, D), jnp.float32),
          jax.ShapeDtypeStruct((N_CORES, npad), jnp.float32),
      ),
      mesh=mesh,
      scratch_types=[
          pltpu.VMEM_SHARED((npad, D), jnp.float32),
          pltpu.VMEM_SHARED((npad,), jnp.float32),
          pltpu.VMEM((2, CHUNK), jnp.int32),
          pltpu.VMEM((2, CHUNK), jnp.int32),
          pltpu.VMEM((2, CHUNK, D), jnp.float32),
          pltpu.VMEM((2, CHUNK, D), jnp.float32),
          pltpu.VMEM((2, CHUNK), jnp.float32),
          pltpu.VMEM((D,), jnp.float32),
          pltpu.SemaphoreType.DMA((2,)),
          pltpu.SemaphoreType.DMA((2,)),
          pltpu.SemaphoreType.DMA((2,)),
      ],
      compiler_params=sc_params,
  )
  acc_parts, asum_parts = phase2(x, h_src, h_dst, src, dst, watt)

  finalize = pl.pallas_call(
      _finalize_body,
      out_shape=jax.ShapeDtypeStruct((N, D), jnp.float32),
  )
  return finalize(acc_parts, asum_parts)
